# baseline (speedup 1.0000x reference)
import functools
import math

import jax
import jax.numpy as jnp
from jax import lax
from jax.experimental import pallas as pl
from jax.experimental.pallas import tpu as pltpu
from jax.experimental.pallas import tpu_sc as plsc

N_FRAG = 16384
FPS = 64
N_CELLS = 4096
N_CLUSTERS = 16
NC, NS, L = 2, 16, 16
NW = NC * NS
B_PER_W = N_FRAG // NW
CH = 128
K_CH = B_PER_W // CH
N_GROUPS = B_PER_W // L
LOG_FPS = math.log(FPS)


def _log_vec(x):
    bits = plsc.bitcast(x, jnp.int32)
    e = ((bits >> 23) & 0xFF) - 127
    m = plsc.bitcast((bits & 0x7FFFFF) | 0x3F800000, jnp.float32)
    big = m >= 1.4142135623730951
    m = jnp.where(big, m * 0.5, m)
    e = e + big.astype(jnp.int32)
    z = (m - 1.0) / (m + 1.0)
    z2 = z * z
    p = 1.0 + z2 * (1.0 / 3.0 + z2 * (0.2 + z2 * (1.0 / 7.0)))
    return e.astype(jnp.float32) * 0.6931471805599453 + 2.0 * z * p


def _body(binc_hbm, gbix_hbm, gpair_hbm, bix_hbm, labels_hbm, cix_hbm,
          table_hbm, diff_hbm,
          out_hbm,
          idx_v, gbix_v, rows_v, binc_v, labels_v, diff_v, cix_v, bix_v,
          y_buf, out_v, sem):
    wid = lax.axis_index("s") * NC + lax.axis_index("c")
    base = wid * B_PER_W

    pltpu.sync_copy(gpair_hbm.at[wid], idx_v)
    gathers = [
        pltpu.async_copy(table_hbm.at[idx_v.at[k]], rows_v.at[k], sem)
        for k in range(K_CH)
    ]
    pltpu.sync_copy(binc_hbm.at[pl.ds(wid * (B_PER_W // 2), B_PER_W // 2)],
                    binc_v)
    pltpu.sync_copy(gbix_hbm.at[pl.ds(base, B_PER_W)], gbix_v)
    pltpu.sync_copy(labels_hbm, labels_v)
    pltpu.sync_copy(diff_hbm, diff_v)
    pltpu.sync_copy(cix_hbm.at[pl.ds(base, B_PER_W)], cix_v)
    pltpu.sync_copy(bix_hbm.at[pl.ds(base, B_PER_W)], bix_v)
    for g in gathers:
        g.wait()

    iota = lax.iota(jnp.int32, L)

    def group_body(g, carry):
        f = g * L + iota
        k_vec = lax.shift_right_logical(f, 7)
        r_vec = f & (CH - 1)
        cix = cix_v[pl.ds(g * L, L)]
        clu = plsc.load_gather(labels_v, [cix])
        w = plsc.load_gather(diff_v, [clu])
        h64 = (gbix_v[pl.ds(g * L, L)] & 1) * FPS
        br_vec = lax.shift_right_logical(f, 1)
        b64 = (f & 1) * FPS

        m = None
        for j in range(FPS):
            bse = plsc.load_gather(rows_v, [k_vec, r_vec, h64 + j])
            bc = plsc.load_gather(binc_v, [br_vec, b64 + j])
            y = bse + jnp.where(bc > 1, w, 0.0)
            y_buf[j >> 3, pl.ds((j & 7) * L, L)] = y
            m = y if m is None else jnp.maximum(m, y)

        s = jnp.zeros((L,), jnp.float32)
        for j in range(FPS):
            s = s + jnp.exp(y_buf[j >> 3, pl.ds((j & 7) * L, L)] - m)

        bflat = bix_v[pl.ds(g * L, L)] * L + iota
        yp = plsc.load_gather(
            y_buf, [lax.shift_right_logical(bflat, 7), bflat & 127])
        out_v[pl.ds(g * L, L)] = yp - m - _log_vec(s) + LOG_FPS
        return carry

    lax.fori_loop(0, N_GROUPS, group_body, 0)
    pltpu.sync_copy(out_v, out_hbm.at[pl.ds(base, B_PER_W)])


@functools.cache
def _make_sc_call():
    mesh = plsc.VectorSubcoreMesh(
        core_axis_name="c", subcore_axis_name="s",
        num_cores=NC, num_subcores=NS)
    return pl.kernel(
        _body,
        out_type=jax.ShapeDtypeStruct((N_FRAG,), jnp.float32),
        mesh=mesh,
        scratch_types=[
            pltpu.VMEM((K_CH, CH), jnp.int32),
            pltpu.VMEM((B_PER_W,), jnp.int32),
            pltpu.VMEM((K_CH, CH, 2 * FPS), jnp.float32),
            pltpu.VMEM((B_PER_W // 2, 2 * FPS), jnp.int32),
            pltpu.VMEM((N_CELLS,), jnp.int32),
            pltpu.VMEM((N_CLUSTERS,), jnp.float32),
            pltpu.VMEM((B_PER_W,), jnp.int32),
            pltpu.VMEM((B_PER_W,), jnp.int32),
            pltpu.VMEM((FPS * L // 128, 128), jnp.float32),
            pltpu.VMEM((B_PER_W,), jnp.float32),
            pltpu.SemaphoreType.DMA,
        ],
        compiler_params=pltpu.CompilerParams(needs_layout_passes=False),
    )


def kernel(bincounts, global_binixs, binixs, labels, local_cell_ix,
           baseline_table, differential_table):
    gbix = global_binixs.reshape(N_FRAG)
    gpair = (gbix >> 1).reshape(NW, K_CH, CH)
    bix = binixs.reshape(N_FRAG)
    diff = differential_table.reshape(N_CLUSTERS)
    table2 = baseline_table.reshape(-1, 2 * FPS)
    binc2 = bincounts.reshape(-1, 2 * FPS)
    return _make_sc_call()(binc2, gbix, gpair, bix, labels, local_cell_ix,
                           table2, diff)


# speedup vs baseline: 1.3482x; 1.3482x over previous
import functools
import math

import jax
import jax.numpy as jnp
from jax import lax
from jax.experimental import pallas as pl
from jax.experimental.pallas import tpu as pltpu
from jax.experimental.pallas import tpu_sc as plsc

N_FRAG = 16384
FPS = 64
N_CELLS = 4096
N_CLUSTERS = 16
NC, NS, L = 2, 16, 16
NW = NC * NS
B_PER_W = N_FRAG // NW
CH = 128
K_CH = B_PER_W // CH
N_GROUPS = B_PER_W // L
LOG_FPS = math.log(FPS)


def _log_vec(x):
    bits = plsc.bitcast(x, jnp.int32)
    e = ((bits >> 23) & 0xFF) - 127
    m = plsc.bitcast((bits & 0x7FFFFF) | 0x3F800000, jnp.float32)
    big = m >= 1.4142135623730951
    m = jnp.where(big, m * 0.5, m)
    e = e + big.astype(jnp.int32)
    z = (m - 1.0) / (m + 1.0)
    z2 = z * z
    p = 1.0 + z2 * (1.0 / 3.0 + z2 * (0.2 + z2 * (1.0 / 7.0)))
    return e.astype(jnp.float32) * 0.6931471805599453 + 2.0 * z * p


def _body(binc_hbm, gbix_hbm, gpair_hbm, bix_hbm, labels_hbm, cix_hbm,
          table_hbm, diff_hbm,
          out_hbm,
          idx_v, gbix_v, rows_v, binc_v, labels_v, diff_v, cix_v, bix_v,
          p_alo, p_blo, p_ahi, p_bhi, out_v, sem):
    wid = lax.axis_index("s") * NC + lax.axis_index("c")
    base = wid * B_PER_W

    pltpu.sync_copy(gpair_hbm.at[wid], idx_v)
    gathers = [
        pltpu.async_copy(table_hbm.at[idx_v.at[k]], rows_v.at[k], sem)
        for k in range(K_CH)
    ]
    pltpu.sync_copy(binc_hbm.at[pl.ds(wid * (B_PER_W // 2), B_PER_W // 2)],
                    binc_v)
    pltpu.sync_copy(gbix_hbm.at[pl.ds(base, B_PER_W)], gbix_v)
    pltpu.sync_copy(labels_hbm, labels_v)
    pltpu.sync_copy(diff_hbm, diff_v)
    pltpu.sync_copy(cix_hbm.at[pl.ds(base, B_PER_W)], cix_v)
    pltpu.sync_copy(bix_hbm.at[pl.ds(base, B_PER_W)], bix_v)
    for g in gathers:
        g.wait()

    iota = lax.iota(jnp.int32, L)
    iota17 = iota * 17

    def group_body(g, carry):
        f = g * L + iota
        cix = cix_v[pl.ds(g * L, L)]
        clu = plsc.load_gather(labels_v, [cix])
        w = plsc.load_gather(diff_v, [clu])
        ew = jnp.exp(w)
        gb = gbix_v[pl.ds(g * L, L)]
        hbit = gb & 1
        bix = bix_v[pl.ds(g * L, L)]

        for i in range(L):
            fi = g * L + i
            kf = lax.shift_right_logical(fi, 7)
            rf = fi & (CH - 1)
            brow = g * (L // 2) + (i >> 1)
            bcol = (i & 1) * FPS
            alo = []
            blo = []
            ahi = []
            bhi = []
            for c in range(FPS // L):
                lo = rows_v[kf, rf, pl.ds(L * c, L)]
                hi = rows_v[kf, rf, pl.ds(FPS + L * c, L)]
                bc = binc_v[brow, pl.ds(bcol + L * c, L)]
                ind = bc > 1
                elo = jnp.exp(lo)
                ehi = jnp.exp(hi)
                zero = jnp.zeros((L,), jnp.float32)
                alo.append(jnp.where(ind, zero, elo))
                blo.append(jnp.where(ind, elo, zero))
                ahi.append(jnp.where(ind, zero, ehi))
                bhi.append(jnp.where(ind, ehi, zero))
            p_alo[pl.ds(i * 17, L)] = (alo[0] + alo[1]) + (alo[2] + alo[3])
            p_blo[pl.ds(i * 17, L)] = (blo[0] + blo[1]) + (blo[2] + blo[3])
            p_ahi[pl.ds(i * 17, L)] = (ahi[0] + ahi[1]) + (ahi[2] + ahi[3])
            p_bhi[pl.ds(i * 17, L)] = (bhi[0] + bhi[1]) + (bhi[2] + bhi[3])

        def colsum(buf):
            t0 = plsc.load_gather(buf, [iota17])
            t1 = plsc.load_gather(buf, [iota17 + 1])
            for b in range(2, L, 2):
                t0 = t0 + plsc.load_gather(buf, [iota17 + b])
                t1 = t1 + plsc.load_gather(buf, [iota17 + b + 1])
            return t0 + t1

        hsel = hbit > 0
        a_sum = jnp.where(hsel, colsum(p_ahi), colsum(p_alo))
        b_sum = jnp.where(hsel, colsum(p_bhi), colsum(p_blo))
        s = a_sum + ew * b_sum

        yp_base = plsc.load_gather(
            rows_v, [lax.shift_right_logical(f, 7), f & (CH - 1),
                     hbit * FPS + bix])
        bcp = plsc.load_gather(
            binc_v, [lax.shift_right_logical(f, 1), (f & 1) * FPS + bix])
        yp = yp_base + jnp.where(bcp > 1, w, 0.0)
        out_v[pl.ds(g * L, L)] = yp - _log_vec(s) + LOG_FPS
        return carry

    lax.fori_loop(0, N_GROUPS, group_body, 0)
    pltpu.sync_copy(out_v, out_hbm.at[pl.ds(base, B_PER_W)])


@functools.cache
def _make_sc_call():
    mesh = plsc.VectorSubcoreMesh(
        core_axis_name="c", subcore_axis_name="s",
        num_cores=NC, num_subcores=NS)
    return pl.kernel(
        _body,
        out_type=jax.ShapeDtypeStruct((N_FRAG,), jnp.float32),
        mesh=mesh,
        scratch_types=[
            pltpu.VMEM((K_CH, CH), jnp.int32),
            pltpu.VMEM((B_PER_W,), jnp.int32),
            pltpu.VMEM((K_CH, CH, 2 * FPS), jnp.float32),
            pltpu.VMEM((B_PER_W // 2, 2 * FPS), jnp.int32),
            pltpu.VMEM((N_CELLS,), jnp.int32),
            pltpu.VMEM((N_CLUSTERS,), jnp.float32),
            pltpu.VMEM((B_PER_W,), jnp.int32),
            pltpu.VMEM((B_PER_W,), jnp.int32),
            pltpu.VMEM((17 * L,), jnp.float32),
            pltpu.VMEM((17 * L,), jnp.float32),
            pltpu.VMEM((17 * L,), jnp.float32),
            pltpu.VMEM((17 * L,), jnp.float32),
            pltpu.VMEM((B_PER_W,), jnp.float32),
            pltpu.SemaphoreType.DMA,
        ],
        compiler_params=pltpu.CompilerParams(needs_layout_passes=False),
    )


def kernel(bincounts, global_binixs, binixs, labels, local_cell_ix,
           baseline_table, differential_table):
    gbix = global_binixs.reshape(N_FRAG)
    gpair = (gbix >> 1).reshape(NW, K_CH, CH)
    bix = binixs.reshape(N_FRAG)
    diff = differential_table.reshape(N_CLUSTERS)
    table2 = baseline_table.reshape(-1, 2 * FPS)
    binc2 = bincounts.reshape(-1, 2 * FPS)
    return _make_sc_call()(binc2, gbix, gpair, bix, labels, local_cell_ix,
                           table2, diff)
